# baseline (speedup 1.0000x reference)
import functools

import jax
import jax.numpy as jnp
from jax.experimental import pallas as pl
import jax.experimental.pallas.tpu as pltpu

NUM_EMBEDDINGS = 8192
EMBEDDING_DIM = 32
DECAY = 0.99
EPSILON = 1e-05

XB = 256
N_ROWS = 8192
N_BLOCKS = N_ROWS // XB


def _vq_stats_kernel(idx_ref, x_ref, cb_ref, ema_count_ref, ema_weight_ref,
                     quant_ref, ec_ref, ew_ref, ncb_ref,
                     cnt_acc, ew_acc):
    i = pl.program_id(0)

    @pl.when(i == 0)
    def _init():
        cnt_acc[...] = jnp.zeros_like(cnt_acc)
        ew_acc[...] = jnp.zeros_like(ew_acc)

    x = x_ref[0]
    cb = cb_ref[...]
    idx = idx_ref[0, 0, :]

    iota = jax.lax.broadcasted_iota(jnp.int32, (XB, NUM_EMBEDDINGS), 1)
    onehot = (iota == idx[:, None]).astype(jnp.float32)

    quant_ref[...] = jnp.dot(onehot, cb)

    cnt_acc[...] += jnp.sum(onehot, axis=0, keepdims=True)
    ew_acc[...] += jnp.dot(onehot.T, x)

    @pl.when(i == N_BLOCKS - 1)
    def _finalize():
        counts = cnt_acc[0, :]
        ec = ema_count_ref[0, :] * DECAY + counts * (1.0 - DECAY)
        n = jnp.sum(ec)
        ec = (ec + EPSILON) / (n + NUM_EMBEDDINGS * EPSILON) * n
        ew = ema_weight_ref[...] * DECAY + ew_acc[...] * (1.0 - DECAY)
        ec_ref[...] = ec[None, :]
        ew_ref[...] = ew
        ncb_ref[...] = ew / ec[:, None]


@functools.partial(jax.jit, static_argnames=("interpret",))
def kernel(x, codebook, ema_count, ema_weight, interpret=False):
    x_flatten = x.reshape(-1, EMBEDDING_DIM)
    distances = (jnp.sum(x_flatten ** 2, axis=1, keepdims=True)
                 + -2 * jnp.dot(x_flatten, codebook.T)
                 + jnp.sum(codebook ** 2, axis=1))
    idx = jnp.argmin(distances, axis=1)
    discrete = jax.nn.one_hot(idx, NUM_EMBEDDINGS)

    idx_b, discrete = jax.lax.optimization_barrier((idx, discrete))
    idx3 = idx_b.astype(jnp.int32).reshape(N_BLOCKS, 1, XB)
    ema_count2 = ema_count.reshape(1, NUM_EMBEDDINGS)

    out_shapes = (
        jax.ShapeDtypeStruct((N_ROWS, EMBEDDING_DIM), jnp.float32),
        jax.ShapeDtypeStruct((1, NUM_EMBEDDINGS), jnp.float32),
        jax.ShapeDtypeStruct((NUM_EMBEDDINGS, EMBEDDING_DIM), jnp.float32),
        jax.ShapeDtypeStruct((NUM_EMBEDDINGS, EMBEDDING_DIM), jnp.float32),
    )

    quant, ec, ew, ncb = pl.pallas_call(
        _vq_stats_kernel,
        grid=(N_BLOCKS,),
        in_specs=[
            pl.BlockSpec((1, 1, XB), lambda i: (i, 0, 0)),
            pl.BlockSpec((1, XB, EMBEDDING_DIM), lambda i: (i // 4, i % 4, 0)),
            pl.BlockSpec((NUM_EMBEDDINGS, EMBEDDING_DIM), lambda i: (0, 0)),
            pl.BlockSpec((1, NUM_EMBEDDINGS), lambda i: (0, 0)),
            pl.BlockSpec((NUM_EMBEDDINGS, EMBEDDING_DIM), lambda i: (0, 0)),
        ],
        out_specs=[
            pl.BlockSpec((XB, EMBEDDING_DIM), lambda i: (i, 0)),
            pl.BlockSpec((1, NUM_EMBEDDINGS), lambda i: (0, 0)),
            pl.BlockSpec((NUM_EMBEDDINGS, EMBEDDING_DIM), lambda i: (0, 0)),
            pl.BlockSpec((NUM_EMBEDDINGS, EMBEDDING_DIM), lambda i: (0, 0)),
        ],
        out_shape=out_shapes,
        scratch_shapes=[
            pltpu.VMEM((1, NUM_EMBEDDINGS), jnp.float32),
            pltpu.VMEM((NUM_EMBEDDINGS, EMBEDDING_DIM), jnp.float32),
        ],
        interpret=interpret,
    )(idx3, x, codebook, ema_count2, ema_weight)

    return (discrete, quant.reshape(x.shape), ec.reshape(NUM_EMBEDDINGS), ew, ncb)


# speedup vs baseline: 1.0331x; 1.0331x over previous
import functools

import jax
import jax.numpy as jnp
from jax.experimental import pallas as pl
import jax.experimental.pallas.tpu as pltpu

NUM_EMBEDDINGS = 8192
EMBEDDING_DIM = 32
DECAY = 0.99
EPSILON = 1e-05

XB = 1024
N_ROWS = 8192
N_BLOCKS = N_ROWS // XB


def _vq_stats_kernel(idx_ref, x_ref, cb_ref, ema_count_ref, ema_weight_ref,
                     quant_ref, ec_ref, ew_ref, ncb_ref,
                     cnt_acc, ew_acc):
    i = pl.program_id(0)

    @pl.when(i == 0)
    def _init():
        cnt_acc[...] = jnp.zeros_like(cnt_acc)
        ew_acc[...] = jnp.zeros_like(ew_acc)

    x = x_ref[0]
    cb = cb_ref[...]
    idx = idx_ref[0, 0, :]

    iota = jax.lax.broadcasted_iota(jnp.int32, (XB, NUM_EMBEDDINGS), 1)
    onehot = (iota == idx[:, None]).astype(jnp.float32)

    quant_ref[...] = jnp.dot(onehot, cb)

    cnt_acc[...] += jnp.sum(onehot, axis=0, keepdims=True)
    ew_acc[...] += jnp.dot(onehot.T, x)

    @pl.when(i == N_BLOCKS - 1)
    def _finalize():
        counts = cnt_acc[0, :]
        ec = ema_count_ref[0, :] * DECAY + counts * (1.0 - DECAY)
        n = jnp.sum(ec)
        ec = (ec + EPSILON) / (n + NUM_EMBEDDINGS * EPSILON) * n
        ew = ema_weight_ref[...] * DECAY + ew_acc[...] * (1.0 - DECAY)
        ec_ref[...] = ec[None, :]
        ew_ref[...] = ew
        ncb_ref[...] = ew / ec[:, None]


@functools.partial(jax.jit, static_argnames=("interpret",))
def kernel(x, codebook, ema_count, ema_weight, interpret=False):
    x_flatten = x.reshape(-1, EMBEDDING_DIM)
    distances = (jnp.sum(x_flatten ** 2, axis=1, keepdims=True)
                 + -2 * jnp.dot(x_flatten, codebook.T)
                 + jnp.sum(codebook ** 2, axis=1))
    idx = jnp.argmin(distances, axis=1)
    discrete = jax.nn.one_hot(idx, NUM_EMBEDDINGS)

    idx_b, discrete = jax.lax.optimization_barrier((idx, discrete))
    idx3 = idx_b.astype(jnp.int32).reshape(N_BLOCKS, 1, XB)
    ema_count2 = ema_count.reshape(1, NUM_EMBEDDINGS)

    out_shapes = (
        jax.ShapeDtypeStruct((N_ROWS, EMBEDDING_DIM), jnp.float32),
        jax.ShapeDtypeStruct((1, NUM_EMBEDDINGS), jnp.float32),
        jax.ShapeDtypeStruct((NUM_EMBEDDINGS, EMBEDDING_DIM), jnp.float32),
        jax.ShapeDtypeStruct((NUM_EMBEDDINGS, EMBEDDING_DIM), jnp.float32),
    )

    quant, ec, ew, ncb = pl.pallas_call(
        _vq_stats_kernel,
        grid=(N_BLOCKS,),
        in_specs=[
            pl.BlockSpec((1, 1, XB), lambda i: (i, 0, 0)),
            pl.BlockSpec((1, XB, EMBEDDING_DIM), lambda i: (i, 0, 0)),
            pl.BlockSpec((NUM_EMBEDDINGS, EMBEDDING_DIM), lambda i: (0, 0)),
            pl.BlockSpec((1, NUM_EMBEDDINGS), lambda i: (0, 0)),
            pl.BlockSpec((NUM_EMBEDDINGS, EMBEDDING_DIM), lambda i: (0, 0)),
        ],
        out_specs=[
            pl.BlockSpec((XB, EMBEDDING_DIM), lambda i: (i, 0)),
            pl.BlockSpec((1, NUM_EMBEDDINGS), lambda i: (0, 0)),
            pl.BlockSpec((NUM_EMBEDDINGS, EMBEDDING_DIM), lambda i: (0, 0)),
            pl.BlockSpec((NUM_EMBEDDINGS, EMBEDDING_DIM), lambda i: (0, 0)),
        ],
        out_shape=out_shapes,
        scratch_shapes=[
            pltpu.VMEM((1, NUM_EMBEDDINGS), jnp.float32),
            pltpu.VMEM((NUM_EMBEDDINGS, EMBEDDING_DIM), jnp.float32),
        ],
        interpret=interpret,
    )(idx3, x, codebook, ema_count2, ema_weight)

    return (discrete, quant.reshape(x.shape), ec.reshape(NUM_EMBEDDINGS), ew, ncb)
